# baseline (speedup 1.0000x reference)
import jax
import jax.numpy as jnp
from jax.experimental import pallas as pl
from jax.experimental.pallas import tpu as pltpu
from jax.experimental.shard_map import shard_map
from jax.sharding import Mesh, PartitionSpec as P
import numpy as np

C = 768
DIM = 768
NSTEPS = 2
BLK = 256


def _flow_kernel(x_ref, v_ref, le_ref, vm_ref,
                 rw1, rb1, rw2, rb2, rw3, rb3, rw4, rb4, rw5, rb5,
                 vw1, vb1, vw2, vb2, vw3, vb3, vw4, vb4, vw5, vb5,
                 v_out, sldj_out):
    x = x_ref[...]
    v = v_ref[...]
    xb = x.astype(jnp.bfloat16)
    epsi = jnp.exp(le_ref[0, 0]) / (2.0 * NSTEPS)

    def dot_t(a, b):
        return jax.lax.dot_general(a.astype(jnp.bfloat16), b,
                                   (((1,), (1,)), ((), ())),
                                   preferred_element_type=jnp.float32)

    def half(i, v_in, m_act, m_upd):
        va = m_act * v_in
        vab = va.astype(jnp.bfloat16)
        xin = jnp.concatenate([xb, vab], axis=1)
        h = jax.nn.relu(dot_t(xin, rw1[i]) + rb1[i])
        h = jax.nn.relu(dot_t(h, rw2[...]) + rb2[...])
        h = jax.nn.relu(dot_t(h, rw3[...]) + rb3[...])
        h = jax.nn.relu(dot_t(h, rw4[...]) + rb4[...])
        grad_e = dot_t(h, rw5[i]) + rb5[i]
        xin2 = jnp.concatenate([xb, vab, grad_e.astype(jnp.bfloat16)],
                               axis=1)
        g = jax.nn.relu(dot_t(xin2, vw1[i]) + vb1[i])
        g = jax.nn.relu(dot_t(g, vw2[...]) + vb2[...])
        g = jax.nn.relu(dot_t(g, vw3[...]) + vb3[...])
        g = jnp.tanh(dot_t(g, vw4[...]) + vb4[...])
        sqt = dot_t(g, vw5[i]) + vb5[i]
        s = sqt[:, :C]
        q = sqt[:, C:2 * C]
        t = sqt[:, 2 * C:]
        v_new = va + m_upd * (v_in * jnp.exp(s)
                              - epsi * (grad_e * jnp.exp(q) + t))
        dsldj = jnp.sum(m_upd * s, axis=1, keepdims=True)
        return v_new, dsldj

    sldj = jnp.zeros((v.shape[0], 1), dtype=jnp.float32)
    for i in range(NSTEPS):
        vm = vm_ref[i]
        vmc = 1.0 - vm
        v, d = half(i, v, vm, vmc)
        sldj = sldj + d
        v, d = half(i, v, vmc, vm)
        sldj = sldj + d
    v_out[...] = v
    sldj_out[...] = sldj


def kernel(x, v, v_mask, log_epsi,
           nv_W1, nv_b1, nv_W2, nv_b2, nv_W3, nv_b3, nv_W4, nv_b4,
           nv_W5, nv_b5,
           nr_W1, nr_b1, nr_W2, nr_b2, nr_W3, nr_b3, nr_W4, nr_b4,
           nr_W5, nr_b5):
    b = x.shape[0]
    f32 = jnp.float32
    bf16 = jnp.bfloat16
    n = NSTEPS

    weights = [
        nr_W1.astype(bf16), nr_b1.reshape(n, 1, DIM),
        nr_W2.astype(bf16), nr_b2.reshape(1, DIM),
        nr_W3.astype(bf16), nr_b3.reshape(1, DIM),
        nr_W4.astype(bf16), nr_b4.reshape(1, DIM),
        nr_W5.astype(bf16), nr_b5.reshape(n, 1, C),
        nv_W1.astype(bf16), nv_b1.reshape(n + 1, 1, DIM),
        nv_W2.astype(bf16), nv_b2.reshape(1, DIM),
        nv_W3.astype(bf16), nv_b3.reshape(1, DIM),
        nv_W4.astype(bf16), nv_b4.reshape(1, DIM),
        nv_W5.astype(bf16), nv_b5.reshape(n + 1, 1, 3 * C),
    ]
    le = log_epsi.reshape(1, 1).astype(f32)
    vm = v_mask.reshape(n, 1, C).astype(f32)

    def run(xs, vs, le_, vm_, *ws):
        bs = xs.shape[0]

        def batch_spec(cols):
            return pl.BlockSpec((BLK, cols), lambda i: (i, 0))

        def head_spec(arr):
            if arr.ndim == 3 and arr.shape[0] > n:
                shape = (n,) + arr.shape[1:]
            else:
                shape = arr.shape
            return pl.BlockSpec(shape, lambda i: (0,) * arr.ndim)

        in_specs = ([batch_spec(C), batch_spec(C),
                     pl.BlockSpec(le_.shape, lambda i: (0, 0)),
                     pl.BlockSpec(vm_.shape, lambda i: (0, 0, 0))]
                    + [head_spec(w) for w in ws])
        return pl.pallas_call(
            _flow_kernel,
            grid=(bs // BLK,),
            in_specs=in_specs,
            out_specs=[batch_spec(C), batch_spec(1)],
            out_shape=[jax.ShapeDtypeStruct((bs, C), f32),
                       jax.ShapeDtypeStruct((bs, 1), f32)],
            compiler_params=pltpu.CompilerParams(
                vmem_limit_bytes=62 * 1024 * 1024),
        )(xs, vs, le_, vm_, *ws)

    devs = jax.devices()
    ndev = len(devs) if b % (BLK * len(devs)) == 0 else 1
    if ndev > 1:
        mesh = Mesh(np.asarray(devs), ("d",))
        rep = (P(),) * (2 + len(weights))
        sharded = shard_map(
            run, mesh=mesh,
            in_specs=(P("d", None), P("d", None)) + rep,
            out_specs=(P("d", None), P("d", None)),
            check_rep=False)
        v_out, sldj = sharded(x, v, le, vm, *weights)
    else:
        v_out, sldj = run(x, v, le, vm, *weights)
    return v_out, sldj.reshape(b)


# speedup vs baseline: 3.9245x; 3.9245x over previous
import jax
import jax.numpy as jnp
from jax.experimental import pallas as pl
from jax.experimental.pallas import tpu as pltpu

C = 768
DIM = 768
NSTEPS = 2
BLK = 512


def _flow_kernel(x_ref, v_ref, le_ref, vm_ref,
                 rw1, rb1, rw2, rb2, rw3, rb3, rw4, rb4, rw5, rb5,
                 vw1, vb1, vw2, vb2, vw3, vb3, vw4, vb4, vw5, vb5,
                 v_out, sldj_out):
    x = x_ref[...]
    v = v_ref[...]
    xb = x.astype(jnp.bfloat16)
    epsi = jnp.exp(le_ref[0, 0]) / (2.0 * NSTEPS)

    def dot_t(a, b):
        return jax.lax.dot_general(a.astype(jnp.bfloat16), b,
                                   (((1,), (1,)), ((), ())),
                                   preferred_element_type=jnp.float32)

    def half(i, v_in, m_act, m_upd):
        va = m_act * v_in
        vab = va.astype(jnp.bfloat16)
        xin = jnp.concatenate([xb, vab], axis=1)
        h = jax.nn.relu(dot_t(xin, rw1[i]) + rb1[i])
        h = jax.nn.relu(dot_t(h, rw2[...]) + rb2[...])
        h = jax.nn.relu(dot_t(h, rw3[...]) + rb3[...])
        h = jax.nn.relu(dot_t(h, rw4[...]) + rb4[...])
        grad_e = dot_t(h, rw5[i]) + rb5[i]
        xin2 = jnp.concatenate([xb, vab, grad_e.astype(jnp.bfloat16)],
                               axis=1)
        g = jax.nn.relu(dot_t(xin2, vw1[i]) + vb1[i])
        g = jax.nn.relu(dot_t(g, vw2[...]) + vb2[...])
        g = jax.nn.relu(dot_t(g, vw3[...]) + vb3[...])
        g = jnp.tanh(dot_t(g, vw4[...]) + vb4[...])
        sqt = dot_t(g, vw5[i]) + vb5[i]
        s = sqt[:, :C]
        q = sqt[:, C:2 * C]
        t = sqt[:, 2 * C:]
        v_new = va + m_upd * (v_in * jnp.exp(s)
                              - epsi * (grad_e * jnp.exp(q) + t))
        dsldj = jnp.sum(m_upd * s, axis=1, keepdims=True)
        return v_new, dsldj

    sldj = jnp.zeros((v.shape[0], 1), dtype=jnp.float32)
    for i in range(NSTEPS):
        vm = vm_ref[i]
        vmc = 1.0 - vm
        v, d = half(i, v, vm, vmc)
        sldj = sldj + d
        v, d = half(i, v, vmc, vm)
        sldj = sldj + d
    v_out[...] = v
    sldj_out[...] = sldj


def kernel(x, v, v_mask, log_epsi,
           nv_W1, nv_b1, nv_W2, nv_b2, nv_W3, nv_b3, nv_W4, nv_b4,
           nv_W5, nv_b5,
           nr_W1, nr_b1, nr_W2, nr_b2, nr_W3, nr_b3, nr_W4, nr_b4,
           nr_W5, nr_b5):
    b = x.shape[0]
    f32 = jnp.float32
    bf16 = jnp.bfloat16
    n = NSTEPS

    weights = [
        nr_W1.astype(bf16), nr_b1.reshape(n, 1, DIM),
        nr_W2.astype(bf16), nr_b2.reshape(1, DIM),
        nr_W3.astype(bf16), nr_b3.reshape(1, DIM),
        nr_W4.astype(bf16), nr_b4.reshape(1, DIM),
        nr_W5.astype(bf16), nr_b5.reshape(n, 1, C),
        nv_W1.astype(bf16), nv_b1.reshape(n + 1, 1, DIM),
        nv_W2.astype(bf16), nv_b2.reshape(1, DIM),
        nv_W3.astype(bf16), nv_b3.reshape(1, DIM),
        nv_W4.astype(bf16), nv_b4.reshape(1, DIM),
        nv_W5.astype(bf16), nv_b5.reshape(n + 1, 1, 3 * C),
    ]
    le = log_epsi.reshape(1, 1).astype(f32)
    vm = v_mask.reshape(n, 1, C).astype(f32)

    def batch_spec(cols):
        return pl.BlockSpec((BLK, cols), lambda i: (i, 0))

    def head_spec(arr):
        if arr.ndim == 3 and arr.shape[0] > n:
            shape = (n,) + arr.shape[1:]
        else:
            shape = arr.shape
        return pl.BlockSpec(shape, lambda i: (0,) * arr.ndim)

    in_specs = ([batch_spec(C), batch_spec(C),
                 pl.BlockSpec(le.shape, lambda i: (0, 0)),
                 pl.BlockSpec(vm.shape, lambda i: (0, 0, 0))]
                + [head_spec(w) for w in weights])
    v_out, sldj = pl.pallas_call(
        _flow_kernel,
        grid=(b // BLK,),
        in_specs=in_specs,
        out_specs=[batch_spec(C), batch_spec(1)],
        out_shape=[jax.ShapeDtypeStruct((b, C), f32),
                   jax.ShapeDtypeStruct((b, 1), f32)],
        compiler_params=pltpu.CompilerParams(
            vmem_limit_bytes=62 * 1024 * 1024),
    )(x, v, le, vm, *weights)
    return v_out, sldj.reshape(b)


# speedup vs baseline: 4.1441x; 1.0560x over previous
import jax
import jax.numpy as jnp
from jax.experimental import pallas as pl
from jax.experimental.pallas import tpu as pltpu

C = 768
DIM = 768
NSTEPS = 2
BLK = 256


def _step_kernel(x_ref, v_ref, sldj_ref, le_ref, vm_ref,
                 rw1, rb1, rw2, rb2, rw3, rb3, rw4, rb4, rw5, rb5,
                 vw1, vb1, vw2, vb2, vw3, vb3, vw4, vb4, vw5, vb5,
                 v_out, sldj_out):
    x = x_ref[...]
    v = v_ref[...]
    epsi = jnp.exp(le_ref[0, 0]) / (2.0 * NSTEPS)

    def dot_t(a, b):
        return jax.lax.dot_general(a, b, (((1,), (1,)), ((), ())),
                                   preferred_element_type=jnp.float32)

    def half(v_in, m_act, m_upd):
        va = m_act * v_in
        xin = jnp.concatenate([x, va], axis=1)
        h = jax.nn.relu(dot_t(xin, rw1[0]) + rb1[0])
        h = jax.nn.relu(dot_t(h, rw2[...]) + rb2[...])
        h = jax.nn.relu(dot_t(h, rw3[...]) + rb3[...])
        h = jax.nn.relu(dot_t(h, rw4[...]) + rb4[...])
        grad_e = dot_t(h, rw5[0]) + rb5[0]
        xin2 = jnp.concatenate([x, va, grad_e], axis=1)
        g = jax.nn.relu(dot_t(xin2, vw1[0]) + vb1[0])
        g = jax.nn.relu(dot_t(g, vw2[...]) + vb2[...])
        g = jax.nn.relu(dot_t(g, vw3[...]) + vb3[...])
        g = jnp.tanh(dot_t(g, vw4[...]) + vb4[...])
        sqt = dot_t(g, vw5[0]) + vb5[0]
        s = sqt[:, :C]
        q = sqt[:, C:2 * C]
        t = sqt[:, 2 * C:]
        v_new = va + m_upd * (v_in * jnp.exp(s)
                              - epsi * (grad_e * jnp.exp(q) + t))
        dsldj = jnp.sum(m_upd * s, axis=1, keepdims=True)
        return v_new, dsldj

    vm = vm_ref[0]
    vmc = 1.0 - vm
    v1, d1 = half(v, vm, vmc)
    v2, d2 = half(v1, vmc, vm)
    v_out[...] = v2
    sldj_out[...] = sldj_ref[...] + d1 + d2


def kernel(x, v, v_mask, log_epsi,
           nv_W1, nv_b1, nv_W2, nv_b2, nv_W3, nv_b3, nv_W4, nv_b4,
           nv_W5, nv_b5,
           nr_W1, nr_b1, nr_W2, nr_b2, nr_W3, nr_b3, nr_W4, nr_b4,
           nr_W5, nr_b5):
    b = x.shape[0]
    f32 = jnp.float32
    n = NSTEPS

    weights = [
        nr_W1, nr_b1.reshape(n, 1, DIM),
        nr_W2, nr_b2.reshape(1, DIM),
        nr_W3, nr_b3.reshape(1, DIM),
        nr_W4, nr_b4.reshape(1, DIM),
        nr_W5, nr_b5.reshape(n, 1, C),
        nv_W1, nv_b1.reshape(n + 1, 1, DIM),
        nv_W2, nv_b2.reshape(1, DIM),
        nv_W3, nv_b3.reshape(1, DIM),
        nv_W4, nv_b4.reshape(1, DIM),
        nv_W5, nv_b5.reshape(n + 1, 1, 3 * C),
    ]
    le = log_epsi.reshape(1, 1).astype(f32)
    vm = v_mask.reshape(n, 1, C).astype(f32)
    sldj = jnp.zeros((b, 1), dtype=f32)

    def batch_spec(cols):
        return pl.BlockSpec((BLK, cols), lambda i: (i, 0))

    def run_step(step, xs, vs, sl):
        def step_spec(arr):
            if arr.ndim == 3:
                return pl.BlockSpec((1,) + arr.shape[1:],
                                    lambda i: (step, 0, 0))
            return pl.BlockSpec(arr.shape, lambda i: (0,) * arr.ndim)

        in_specs = ([batch_spec(C), batch_spec(C), batch_spec(1),
                     pl.BlockSpec(le.shape, lambda i: (0, 0)),
                     pl.BlockSpec((1, 1, C), lambda i: (step, 0, 0))]
                    + [step_spec(w) for w in weights])
        return pl.pallas_call(
            _step_kernel,
            grid=(b // BLK,),
            in_specs=in_specs,
            out_specs=[batch_spec(C), batch_spec(1)],
            out_shape=[jax.ShapeDtypeStruct((b, C), f32),
                       jax.ShapeDtypeStruct((b, 1), f32)],
            compiler_params=pltpu.CompilerParams(
                vmem_limit_bytes=62 * 1024 * 1024),
        )(xs, vs, sl, le, vm, *weights)

    for i in range(n):
        v, sldj = run_step(i, x, v, sldj)
    return v, sldj.reshape(b)
